# baseline (speedup 1.0000x reference)
import functools

import jax
import jax.numpy as jnp
from jax import lax
from jax.experimental import pallas as pl
from jax.experimental.pallas import tpu as pltpu
from jax.experimental.pallas import tpu_sc as plsc

N = 10000
E = 320000
D_FEAT = 128
HIDDEN = 16
NUM_CLASSES = 10

NC = 2
NS = 16
NW = NC * NS
N2 = 10240
RPT = N2 // NS
EPW = E // NW
K = 2000
NB = EPW // K
BLK = 1000


def _sc_mesh():
    return plsc.VectorSubcoreMesh(core_axis_name="c", subcore_axis_name="s")


def _degree(col):
    @functools.partial(
        pl.kernel,
        out_type=jax.ShapeDtypeStruct((NC, N2, 128), jnp.float32),
        mesh=_sc_mesh(),
        compiler_params=pltpu.CompilerParams(use_tc_tiling_on_sc=False),
        scratch_types=[
            pltpu.VMEM_SHARED((N2,), jnp.float32),
            pltpu.VMEM((RPT,), jnp.float32),
            pltpu.VMEM((RPT, HIDDEN), jnp.float32),
            pltpu.VMEM((K,), jnp.int32),
            pltpu.VMEM((K,), jnp.int32),
            pltpu.SemaphoreType.DMA,
            pltpu.SemaphoreType.DMA,
            pltpu.VMEM((K,), jnp.float32),
        ],
    )
    def deg_kernel(col_hbm, out_hbm, hist_sp, zbuf, dbuf, cidx0, cidx1,
                   csem0, csem1, ones_v):
        cid = lax.axis_index("c")
        sid = lax.axis_index("s")
        wid = sid * NC + cid
        nsl = pl.ds(sid * RPT, RPT)

        def fill_zero(i, _):
            zbuf[pl.ds(i * 16, 16)] = jnp.zeros((16,), jnp.float32)
            return 0

        lax.fori_loop(0, RPT // 16, fill_zero, 0)

        def fill_one(i, _):
            ones_v[pl.ds(i * 16, 16)] = jnp.ones((16,), jnp.float32)
            return 0

        lax.fori_loop(0, K // 16, fill_one, 0)

        pltpu.sync_copy(zbuf, hist_sp.at[nsl])
        plsc.subcore_barrier()

        base = wid * EPW
        bufs = ((cidx0, csem0), (cidx1, csem1))

        def start(i, b):
            cidx, csem = bufs[b]
            return pltpu.async_copy(col_hbm.at[pl.ds(base + i * K, K)], cidx, csem)

        d = start(0, 0)
        for i in range(NB):
            b = i % 2
            d.wait()
            if i + 1 < NB:
                d = start(i + 1, (i + 1) % 2)
            pltpu.sync_copy(ones_v, hist_sp.at[bufs[b][0]], add=True)

        plsc.subcore_barrier()

        pltpu.sync_copy(hist_sp.at[nsl], zbuf)

        def splat(g, _):
            v = zbuf[pl.ds(g * 16, 16)]
            for j in range(16):
                dbuf[g * 16 + j, :] = jnp.full((HIDDEN,), v[j], jnp.float32)
            return 0

        lax.fori_loop(0, RPT // 16, splat, 0)
        pltpu.sync_copy(dbuf, out_hbm.at[cid, nsl, pl.ds(0, HIDDEN)])

    return deg_kernel(col)


def _propagate(y_pad, row, col):
    @functools.partial(
        pl.kernel,
        out_type=jax.ShapeDtypeStruct((NC, N2, 128), jnp.float32),
        mesh=_sc_mesh(),
        compiler_params=pltpu.CompilerParams(use_tc_tiling_on_sc=False),
        scratch_types=[
            pltpu.VMEM_SHARED((N2, HIDDEN), jnp.float32),
            pltpu.VMEM_SHARED((N2, HIDDEN), jnp.float32),
            pltpu.VMEM((RPT, HIDDEN), jnp.float32),
            pltpu.VMEM((K,), jnp.int32),
            pltpu.VMEM((K,), jnp.int32),
            pltpu.VMEM((K, HIDDEN), jnp.float32),
            pltpu.SemaphoreType.DMA,
            pltpu.VMEM((K,), jnp.int32),
            pltpu.VMEM((K,), jnp.int32),
            pltpu.VMEM((K, HIDDEN), jnp.float32),
            pltpu.SemaphoreType.DMA,
        ],
    )
    def prop_kernel(y_hbm, row_hbm, col_hbm, out_hbm, y_sp, acc_sp, stage,
                    ridx0, cidx0, msg0, sem0, ridx1, cidx1, msg1, sem1):
        cid = lax.axis_index("c")
        sid = lax.axis_index("s")
        wid = sid * NC + cid
        nsl = pl.ds(sid * RPT, RPT)
        lanes = pl.ds(0, HIDDEN)

        pltpu.sync_copy(y_hbm.at[nsl, lanes], stage)
        pltpu.sync_copy(stage, y_sp.at[nsl])
        pltpu.sync_copy(stage, acc_sp.at[nsl])
        plsc.subcore_barrier()

        base = wid * EPW
        bufs = ((ridx0, cidx0, msg0, sem0), (ridx1, cidx1, msg1, sem1))

        def start(i, b):
            ridx, cidx, msg, sem = bufs[b]
            pltpu.sync_copy(row_hbm.at[pl.ds(base + i * K, K)], ridx)
            pltpu.sync_copy(col_hbm.at[pl.ds(base + i * K, K)], cidx)
            return pltpu.async_copy(y_sp.at[ridx], msg, sem)

        d = start(0, 0)
        for i in range(NB):
            b = i % 2
            d.wait()
            if i + 1 < NB:
                d = start(i + 1, (i + 1) % 2)
            pltpu.sync_copy(bufs[b][2], acc_sp.at[bufs[b][1]], add=True)

        plsc.subcore_barrier()
        pltpu.sync_copy(acc_sp.at[nsl], out_hbm.at[cid, nsl, lanes])

    return prop_kernel(y_pad, row, col)


def _row128(i):
    return (i, 0)


def _prep1(x, W1, W2, hist):
    def body(x_ref, w1_ref, w2_ref, h0_ref, h1_ref, y_ref, dmat_ref, w2x_ref):
        deg = (h0_ref[0][:, :HIDDEN] + h1_ref[0][:, :HIDDEN] + 1.0)
        dis = lax.rsqrt(deg)
        xw = jnp.dot(x_ref[...], w1_ref[...], preferred_element_type=jnp.float32)
        y = xw * dis
        y_ref[...] = jnp.concatenate(
            [y, jnp.zeros((BLK, 128 - HIDDEN), jnp.float32)], axis=1
        )
        dmat_ref[...] = jnp.concatenate([dis] * 8, axis=1)

        @pl.when(pl.program_id(0) == 0)
        def _():
            w2x_ref[...] = jnp.zeros((HIDDEN, 128), jnp.float32)
            w2x_ref[0:HIDDEN, 0:NUM_CLASSES] = w2_ref[...]

    return pl.pallas_call(
        body,
        grid=(N // BLK,),
        in_specs=[
            pl.BlockSpec((BLK, D_FEAT), _row128),
            pl.BlockSpec((D_FEAT, HIDDEN), lambda i: (0, 0)),
            pl.BlockSpec((HIDDEN, NUM_CLASSES), lambda i: (0, 0)),
            pl.BlockSpec((1, BLK, 128), lambda i: (0, i, 0)),
            pl.BlockSpec((1, BLK, 128), lambda i: (1, i, 0)),
        ],
        out_specs=[
            pl.BlockSpec((BLK, 128), _row128),
            pl.BlockSpec((BLK, 128), _row128),
            pl.BlockSpec((HIDDEN, 128), lambda i: (0, 0)),
        ],
        out_shape=[
            jax.ShapeDtypeStruct((N2, 128), jnp.float32),
            jax.ShapeDtypeStruct((N2, 128), jnp.float32),
            jax.ShapeDtypeStruct((HIDDEN, 128), jnp.float32),
        ],
    )(x, W1, W2, hist, hist)


def _mid(acc, y1, dmat, b1p, w2x):
    def body(a0_ref, a1_ref, y_ref, d_ref, b_ref, w_ref, y2_ref):
        a01 = a0_ref[0][:, :HIDDEN] + a1_ref[0][:, :HIDDEN]
        h = (d_ref[...][:, :HIDDEN] * (a01 - y_ref[...][:, :HIDDEN])
             + b_ref[...][:, :HIDDEN])
        h = jnp.maximum(h, 0.0)
        y2_ref[...] = (
            jnp.dot(h, w_ref[...], preferred_element_type=jnp.float32) * d_ref[...]
        )

    return pl.pallas_call(
        body,
        grid=(N // BLK,),
        in_specs=[
            pl.BlockSpec((1, BLK, 128), lambda i: (0, i, 0)),
            pl.BlockSpec((1, BLK, 128), lambda i: (1, i, 0)),
            pl.BlockSpec((BLK, 128), _row128),
            pl.BlockSpec((BLK, 128), _row128),
            pl.BlockSpec((1, 128), lambda i: (0, 0)),
            pl.BlockSpec((HIDDEN, 128), lambda i: (0, 0)),
        ],
        out_specs=pl.BlockSpec((BLK, 128), _row128),
        out_shape=jax.ShapeDtypeStruct((N2, 128), jnp.float32),
    )(acc, acc, y1, dmat, b1p, w2x)


def _final(acc, y2, dmat, b2p):
    def body(a0_ref, a1_ref, y_ref, d_ref, b_ref, o_ref):
        a01 = a0_ref[0][:, :HIDDEN] + a1_ref[0][:, :HIDDEN]
        o = d_ref[...][:, :HIDDEN] * (a01 - y_ref[...][:, :HIDDEN])
        o10 = o[:, :NUM_CLASSES] + b_ref[...][:, :NUM_CLASSES]
        m = jnp.max(o10, axis=1, keepdims=True)
        z = o10 - m
        lse = jnp.log(jnp.sum(jnp.exp(z), axis=1, keepdims=True))
        o_ref[...] = z - lse

    return pl.pallas_call(
        body,
        grid=(N // BLK,),
        in_specs=[
            pl.BlockSpec((1, BLK, 128), lambda i: (0, i, 0)),
            pl.BlockSpec((1, BLK, 128), lambda i: (1, i, 0)),
            pl.BlockSpec((BLK, 128), _row128),
            pl.BlockSpec((BLK, 128), _row128),
            pl.BlockSpec((1, 128), lambda i: (0, 0)),
        ],
        out_specs=pl.BlockSpec((BLK, NUM_CLASSES), _row128),
        out_shape=jax.ShapeDtypeStruct((N, NUM_CLASSES), jnp.float32),
    )(acc, acc, y2, dmat, b2p)


def kernel(x, edge_index, W1, b1, W2, b2):
    row = edge_index[0]
    col = edge_index[1]
    b1p = jnp.pad(b1, (0, 128 - HIDDEN)).reshape(1, 128)
    b2p = jnp.pad(b2, (0, 128 - NUM_CLASSES)).reshape(1, 128)

    hist = _degree(col)
    y1, dmat, w2x = _prep1(x, W1, W2, hist)
    acc = _propagate(y1, row, col)
    y2 = _mid(acc, y1, dmat, b1p, w2x)
    acc2 = _propagate(y2, row, col)
    return _final(acc2, y2, dmat, b2p)


# speedup vs baseline: 1.0274x; 1.0274x over previous
import functools

import jax
import jax.numpy as jnp
from jax import lax
from jax.experimental import pallas as pl
from jax.experimental.pallas import tpu as pltpu
from jax.experimental.pallas import tpu_sc as plsc

N = 10000
E = 320000
D_FEAT = 128
HIDDEN = 16
NUM_CLASSES = 10

NC = 2
NS = 16
NW = NC * NS
N2 = 10240
RPT = N2 // NS
EPW = E // NW
K = 2000
NB = EPW // K
BLK = 1000


def _sc_mesh():
    return plsc.VectorSubcoreMesh(core_axis_name="c", subcore_axis_name="s")


def _degree(col):
    @functools.partial(
        pl.kernel,
        out_type=jax.ShapeDtypeStruct((NC, N2, 128), jnp.float32),
        mesh=_sc_mesh(),
        compiler_params=pltpu.CompilerParams(use_tc_tiling_on_sc=False),
        scratch_types=[
            pltpu.VMEM_SHARED((N2,), jnp.float32),
            pltpu.VMEM((RPT,), jnp.float32),
            pltpu.VMEM((RPT, HIDDEN), jnp.float32),
            pltpu.VMEM((K,), jnp.int32),
            pltpu.VMEM((K,), jnp.int32),
            pltpu.SemaphoreType.DMA,
            pltpu.SemaphoreType.DMA,
            pltpu.VMEM((K,), jnp.float32),
        ],
    )
    def deg_kernel(col_hbm, out_hbm, hist_sp, zbuf, dbuf, cidx0, cidx1,
                   csem0, csem1, ones_v):
        cid = lax.axis_index("c")
        sid = lax.axis_index("s")
        wid = sid * NC + cid
        nsl = pl.ds(sid * RPT, RPT)

        def fill_zero(i, _):
            zbuf[pl.ds(i * 16, 16)] = jnp.zeros((16,), jnp.float32)
            return 0

        lax.fori_loop(0, RPT // 16, fill_zero, 0)

        def fill_one(i, _):
            ones_v[pl.ds(i * 16, 16)] = jnp.ones((16,), jnp.float32)
            return 0

        lax.fori_loop(0, K // 16, fill_one, 0)

        pltpu.sync_copy(zbuf, hist_sp.at[nsl])
        plsc.subcore_barrier()

        base = wid * EPW
        bufs = ((cidx0, csem0), (cidx1, csem1))

        def start(i, b):
            cidx, csem = bufs[b]
            return pltpu.async_copy(col_hbm.at[pl.ds(base + i * K, K)], cidx, csem)

        d = start(0, 0)
        for i in range(NB):
            b = i % 2
            d.wait()
            if i + 1 < NB:
                d = start(i + 1, (i + 1) % 2)
            pltpu.sync_copy(ones_v, hist_sp.at[bufs[b][0]], add=True)

        plsc.subcore_barrier()

        pltpu.sync_copy(hist_sp.at[nsl], zbuf)

        def splat(g, _):
            v = zbuf[pl.ds(g * 16, 16)]
            for j in range(16):
                dbuf[g * 16 + j, :] = jnp.full((HIDDEN,), v[j], jnp.float32)
            return 0

        lax.fori_loop(0, RPT // 16, splat, 0)
        pltpu.sync_copy(dbuf, out_hbm.at[cid, nsl, pl.ds(0, HIDDEN)])

    return deg_kernel(col)


def _propagate(y_pad, row, col):
    @functools.partial(
        pl.kernel,
        out_type=jax.ShapeDtypeStruct((NC, N2, 128), jnp.float32),
        mesh=_sc_mesh(),
        compiler_params=pltpu.CompilerParams(use_tc_tiling_on_sc=False),
        scratch_types=[
            pltpu.VMEM_SHARED((N2, HIDDEN), jnp.float32),
            pltpu.VMEM_SHARED((N2, HIDDEN), jnp.float32),
            pltpu.VMEM((RPT, HIDDEN), jnp.float32),
            pltpu.VMEM((K,), jnp.int32),
            pltpu.VMEM((K,), jnp.int32),
            pltpu.VMEM((K, HIDDEN), jnp.float32),
            pltpu.SemaphoreType.DMA,
            pltpu.VMEM((K,), jnp.int32),
            pltpu.VMEM((K,), jnp.int32),
            pltpu.VMEM((K, HIDDEN), jnp.float32),
            pltpu.SemaphoreType.DMA,
        ],
    )
    def prop_kernel(y_hbm, row_hbm, col_hbm, out_hbm, y_sp, acc_sp, stage,
                    ridx0, cidx0, msg0, sem0, ridx1, cidx1, msg1, sem1):
        cid = lax.axis_index("c")
        sid = lax.axis_index("s")
        wid = sid * NC + cid
        nsl = pl.ds(sid * RPT, RPT)
        lanes = pl.ds(0, HIDDEN)

        pltpu.sync_copy(y_hbm.at[nsl, lanes], stage)
        pltpu.sync_copy(stage, y_sp.at[nsl])
        pltpu.sync_copy(stage, acc_sp.at[nsl])
        plsc.subcore_barrier()

        base = wid * EPW
        bufs = ((ridx0, cidx0, msg0, sem0), (ridx1, cidx1, msg1, sem1))

        def start(i, b):
            ridx, cidx, msg, sem = bufs[b]
            pltpu.sync_copy(row_hbm.at[pl.ds(base + i * K, K)], ridx)
            pltpu.sync_copy(col_hbm.at[pl.ds(base + i * K, K)], cidx)
            return pltpu.async_copy(y_sp.at[ridx], msg, sem)

        d = start(0, 0)
        for i in range(NB):
            b = i % 2
            d.wait()
            if i + 1 < NB:
                d = start(i + 1, (i + 1) % 2)
            pltpu.sync_copy(bufs[b][2], acc_sp.at[bufs[b][1]], add=True)

        plsc.subcore_barrier()
        pltpu.sync_copy(acc_sp.at[nsl], out_hbm.at[cid, nsl, lanes])

    return prop_kernel(y_pad, row, col)


def _row128(i):
    return (i, 0)


def _prep1(x, W1, W2, hist):
    def body(x_ref, w1_ref, w2_ref, h0_ref, h1_ref, y_ref, dmat_ref, w2x_ref):
        deg = h0_ref[0] + h1_ref[0] + 1.0
        dis = lax.rsqrt(deg)
        xw = jnp.dot(x_ref[...], w1_ref[...], preferred_element_type=jnp.float32)
        y = xw * dis[:, :HIDDEN]
        y_ref[...] = jnp.concatenate(
            [y, jnp.zeros((BLK, 128 - HIDDEN), jnp.float32)], axis=1
        )
        dmat_ref[...] = dis

        @pl.when(pl.program_id(0) == 0)
        def _():
            w2x_ref[...] = jnp.zeros((128, 128), jnp.float32)
            w2x_ref[0:HIDDEN, 0:NUM_CLASSES] = w2_ref[...]

    return pl.pallas_call(
        body,
        grid=(N // BLK,),
        in_specs=[
            pl.BlockSpec((BLK, D_FEAT), _row128),
            pl.BlockSpec((D_FEAT, HIDDEN), lambda i: (0, 0)),
            pl.BlockSpec((HIDDEN, NUM_CLASSES), lambda i: (0, 0)),
            pl.BlockSpec((1, BLK, 128), lambda i: (0, i, 0)),
            pl.BlockSpec((1, BLK, 128), lambda i: (1, i, 0)),
        ],
        out_specs=[
            pl.BlockSpec((BLK, 128), _row128),
            pl.BlockSpec((BLK, 128), _row128),
            pl.BlockSpec((128, 128), lambda i: (0, 0)),
        ],
        out_shape=[
            jax.ShapeDtypeStruct((N2, 128), jnp.float32),
            jax.ShapeDtypeStruct((N2, 128), jnp.float32),
            jax.ShapeDtypeStruct((128, 128), jnp.float32),
        ],
    )(x, W1, W2, hist, hist)


def _mid(acc, y1, dmat, b1p, w2x):
    def body(a0_ref, a1_ref, y_ref, d_ref, b_ref, w_ref, y2_ref):
        lane = lax.broadcasted_iota(jnp.int32, (BLK, 128), 1)
        h = d_ref[...] * (a0_ref[0] + a1_ref[0] - y_ref[...]) + b_ref[...]
        h = jnp.maximum(h, 0.0)
        h = jnp.where(lane < HIDDEN, h, 0.0)
        y2_ref[...] = (
            jnp.dot(h, w_ref[...], preferred_element_type=jnp.float32) * d_ref[...]
        )

    return pl.pallas_call(
        body,
        grid=(N // BLK,),
        in_specs=[
            pl.BlockSpec((1, BLK, 128), lambda i: (0, i, 0)),
            pl.BlockSpec((1, BLK, 128), lambda i: (1, i, 0)),
            pl.BlockSpec((BLK, 128), _row128),
            pl.BlockSpec((BLK, 128), _row128),
            pl.BlockSpec((1, 128), lambda i: (0, 0)),
            pl.BlockSpec((128, 128), lambda i: (0, 0)),
        ],
        out_specs=pl.BlockSpec((BLK, 128), _row128),
        out_shape=jax.ShapeDtypeStruct((N2, 128), jnp.float32),
    )(acc, acc, y1, dmat, b1p, w2x)


def _final(acc, y2, dmat, b2p):
    def body(a0_ref, a1_ref, y_ref, d_ref, b_ref, o_ref):
        o = d_ref[...] * (a0_ref[0] + a1_ref[0] - y_ref[...])
        o10 = o[:, :NUM_CLASSES] + b_ref[...][:, :NUM_CLASSES]
        m = jnp.max(o10, axis=1, keepdims=True)
        z = o10 - m
        lse = jnp.log(jnp.sum(jnp.exp(z), axis=1, keepdims=True))
        o_ref[...] = z - lse

    return pl.pallas_call(
        body,
        grid=(N // BLK,),
        in_specs=[
            pl.BlockSpec((1, BLK, 128), lambda i: (0, i, 0)),
            pl.BlockSpec((1, BLK, 128), lambda i: (1, i, 0)),
            pl.BlockSpec((BLK, 128), _row128),
            pl.BlockSpec((BLK, 128), _row128),
            pl.BlockSpec((1, 128), lambda i: (0, 0)),
        ],
        out_specs=pl.BlockSpec((BLK, NUM_CLASSES), _row128),
        out_shape=jax.ShapeDtypeStruct((N, NUM_CLASSES), jnp.float32),
    )(acc, acc, y2, dmat, b2p)


def kernel(x, edge_index, W1, b1, W2, b2):
    row = edge_index[0]
    col = edge_index[1]
    b1p = jnp.pad(b1, (0, 128 - HIDDEN)).reshape(1, 128)
    b2p = jnp.pad(b2, (0, 128 - NUM_CLASSES)).reshape(1, 128)

    hist = _degree(col)
    y1, dmat, w2x = _prep1(x, W1, W2, hist)
    acc = _propagate(y1, row, col)
    y2 = _mid(acc, y1, dmat, b1p, w2x)
    acc2 = _propagate(y2, row, col)
    return _final(acc2, y2, dmat, b2p)
